# baseline (device time: 28377 ns/iter reference)
import jax
import jax.numpy as jnp
from jax import lax
from jax.experimental import pallas as pl
from jax.experimental.pallas import tpu as pltpu

BM = 512


def kernel(x, dy, gamma):
    del gamma
    m, d = x.shape
    n_steps = m // BM

    def body(x_ref, dy_ref, out_ref, acc_ref, recv_ref, send_sem, recv_sem):
        step = pl.program_id(0)
        my_x = lax.axis_index("x")
        my_y = lax.axis_index("y")
        nbr = (1 - my_x, my_y)

        @pl.when(step == 0)
        def _init():
            acc_ref[...] = jnp.zeros_like(acc_ref)

        xb = x_ref[...]
        dyb = dy_ref[...]
        mu = jnp.mean(xb, axis=1, keepdims=True)
        xc = xb - mu
        var = jnp.mean(xc * xc, axis=1, keepdims=True)
        rstd = lax.rsqrt(var + 1e-5)
        xhat = xc * rstd
        acc_ref[0, :] = acc_ref[0, :] + jnp.sum(dyb * xhat, axis=0)
        acc_ref[1, :] = acc_ref[1, :] + jnp.sum(dyb, axis=0)

        @pl.when(step == n_steps - 1)
        def _exchange():
            barrier = pltpu.get_barrier_semaphore()
            pl.semaphore_signal(
                barrier,
                inc=1,
                device_id=nbr,
                device_id_type=pl.DeviceIdType.MESH,
            )
            pl.semaphore_wait(barrier, 1)

            rdma = pltpu.make_async_remote_copy(
                src_ref=acc_ref,
                dst_ref=recv_ref,
                send_sem=send_sem,
                recv_sem=recv_sem,
                device_id=nbr,
                device_id_type=pl.DeviceIdType.MESH,
            )
            rdma.start()
            rdma.wait()
            out_ref[...] = acc_ref[...] + recv_ref[...]

    return pl.pallas_call(
        body,
        grid=(n_steps,),
        in_specs=[
            pl.BlockSpec((BM, d), lambda i: (i, 0)),
            pl.BlockSpec((BM, d), lambda i: (i, 0)),
        ],
        out_specs=pl.BlockSpec((2, d), lambda i: (0, 0)),
        out_shape=jax.ShapeDtypeStruct((2, d), jnp.float32),
        scratch_shapes=[
            pltpu.VMEM((2, d), jnp.float32),
            pltpu.VMEM((2, d), jnp.float32),
            pltpu.SemaphoreType.DMA,
            pltpu.SemaphoreType.DMA,
        ],
        compiler_params=pltpu.CompilerParams(
            collective_id=0, vmem_limit_bytes=96 * 1024 * 1024
        ),
    )(x, dy)


# device time: 20977 ns/iter; 1.3528x vs baseline; 1.3528x over previous
import jax
import jax.numpy as jnp
from jax import lax
from jax.experimental import pallas as pl
from jax.experimental.pallas import tpu as pltpu

BM = 512


def kernel(x, dy, gamma):
    del gamma
    m, d = x.shape
    half_steps = (m // 2) // BM

    def body(off_ref, x_ref, dy_ref, out_ref, acc_ref, recv_ref,
             send_sems, recv_sems):
        step = pl.program_id(0)
        my_x = lax.axis_index("x")
        my_y = lax.axis_index("y")
        y_nbr = (my_x, 1 - my_y)
        x_nbr = (1 - my_x, my_y)

        @pl.when(step == 0)
        def _init():
            acc_ref[...] = jnp.zeros_like(acc_ref)

        xb = x_ref[...]
        dyb = dy_ref[...]
        mu = jnp.mean(xb, axis=1, keepdims=True)
        xc = xb - mu
        var = jnp.mean(xc * xc, axis=1, keepdims=True)
        rstd = lax.rsqrt(var + 1e-5)
        xhat = xc * rstd
        acc_ref[0, :] = acc_ref[0, :] + jnp.sum(dyb * xhat, axis=0)
        acc_ref[1, :] = acc_ref[1, :] + jnp.sum(dyb, axis=0)

        @pl.when(step == half_steps - 1)
        def _allreduce():
            barrier = pltpu.get_barrier_semaphore()
            for nbr in (y_nbr, x_nbr):
                pl.semaphore_signal(
                    barrier,
                    inc=1,
                    device_id=nbr,
                    device_id_type=pl.DeviceIdType.MESH,
                )
            pl.semaphore_wait(barrier, 2)

            rdma1 = pltpu.make_async_remote_copy(
                src_ref=acc_ref,
                dst_ref=recv_ref.at[0],
                send_sem=send_sems.at[0],
                recv_sem=recv_sems.at[0],
                device_id=y_nbr,
                device_id_type=pl.DeviceIdType.MESH,
            )
            rdma1.start()
            rdma1.wait()
            acc_ref[...] = acc_ref[...] + recv_ref[0]

            rdma2 = pltpu.make_async_remote_copy(
                src_ref=acc_ref,
                dst_ref=recv_ref.at[1],
                send_sem=send_sems.at[1],
                recv_sem=recv_sems.at[1],
                device_id=x_nbr,
                device_id_type=pl.DeviceIdType.MESH,
            )
            rdma2.start()
            rdma2.wait()
            out_ref[...] = acc_ref[...] + recv_ref[1]

    grid_spec = pltpu.PrefetchScalarGridSpec(
        num_scalar_prefetch=1,
        grid=(half_steps,),
        in_specs=[
            pl.BlockSpec((BM, d), lambda i, off: (off[0] + i, 0)),
            pl.BlockSpec((BM, d), lambda i, off: (off[0] + i, 0)),
        ],
        out_specs=pl.BlockSpec((2, d), lambda i, off: (0, 0)),
        scratch_shapes=[
            pltpu.VMEM((2, d), jnp.float32),
            pltpu.VMEM((2, 2, d), jnp.float32),
            pltpu.SemaphoreType.DMA((2,)),
            pltpu.SemaphoreType.DMA((2,)),
        ],
    )

    offset = (lax.axis_index("y") * half_steps).astype(jnp.int32).reshape(1)

    return pl.pallas_call(
        body,
        grid_spec=grid_spec,
        out_shape=jax.ShapeDtypeStruct((2, d), jnp.float32),
        compiler_params=pltpu.CompilerParams(
            collective_id=0, vmem_limit_bytes=96 * 1024 * 1024
        ),
    )(offset, x, dy)


# device time: 19678 ns/iter; 1.4421x vs baseline; 1.0660x over previous
import jax
import jax.numpy as jnp
from jax import lax
from jax.experimental import pallas as pl
from jax.experimental.pallas import tpu as pltpu

BM = 512


def kernel(x, dy, gamma):
    del gamma
    m, d = x.shape
    half_steps = (m // 2) // BM

    def body(off_ref, x_ref, dy_ref, out_ref, acc_ref, recv_ref,
             send_sems, recv_sems):
        step = pl.program_id(0)
        my_x = lax.axis_index("x")
        my_y = lax.axis_index("y")
        peers = [
            (my_x, 1 - my_y),
            (1 - my_x, my_y),
            (1 - my_x, 1 - my_y),
        ]

        @pl.when(step == 0)
        def _init():
            acc_ref[...] = jnp.zeros_like(acc_ref)
            barrier = pltpu.get_barrier_semaphore()
            for nbr in peers:
                pl.semaphore_signal(
                    barrier,
                    inc=1,
                    device_id=nbr,
                    device_id_type=pl.DeviceIdType.MESH,
                )
            pl.semaphore_wait(barrier, 3)

        xb = x_ref[...]
        dyb = dy_ref[...]
        mu = jnp.mean(xb, axis=1, keepdims=True)
        xc = xb - mu
        var = jnp.mean(xc * xc, axis=1, keepdims=True)
        rstd = lax.rsqrt(var + 1e-5)
        xhat = xc * rstd
        acc_ref[0, :] = acc_ref[0, :] + jnp.sum(dyb * xhat, axis=0)
        acc_ref[1, :] = acc_ref[1, :] + jnp.sum(dyb, axis=0)

        @pl.when(step == half_steps - 1)
        def _allreduce():
            rdmas = []
            for k, nbr in enumerate(peers):
                rdma = pltpu.make_async_remote_copy(
                    src_ref=acc_ref,
                    dst_ref=recv_ref.at[k],
                    send_sem=send_sems.at[k],
                    recv_sem=recv_sems.at[k],
                    device_id=nbr,
                    device_id_type=pl.DeviceIdType.MESH,
                )
                rdma.start()
                rdmas.append(rdma)
            for rdma in rdmas:
                rdma.wait()
            out_ref[...] = (
                acc_ref[...] + recv_ref[0] + recv_ref[1] + recv_ref[2]
            )

    grid_spec = pltpu.PrefetchScalarGridSpec(
        num_scalar_prefetch=1,
        grid=(half_steps,),
        in_specs=[
            pl.BlockSpec((BM, d), lambda i, off: (off[0] + i, 0)),
            pl.BlockSpec((BM, d), lambda i, off: (off[0] + i, 0)),
        ],
        out_specs=pl.BlockSpec((2, d), lambda i, off: (0, 0)),
        scratch_shapes=[
            pltpu.VMEM((2, d), jnp.float32),
            pltpu.VMEM((3, 2, d), jnp.float32),
            pltpu.SemaphoreType.DMA((3,)),
            pltpu.SemaphoreType.DMA((3,)),
        ],
    )

    offset = (lax.axis_index("y") * half_steps).astype(jnp.int32).reshape(1)

    return pl.pallas_call(
        body,
        grid_spec=grid_spec,
        out_shape=jax.ShapeDtypeStruct((2, d), jnp.float32),
        compiler_params=pltpu.CompilerParams(
            collective_id=0, vmem_limit_bytes=96 * 1024 * 1024
        ),
    )(offset, x, dy)
